# Initial kernel scaffold; baseline (speedup 1.0000x reference)
import jax
import jax.numpy as jnp
from jax.experimental import pallas as pl


def kernel(x, edge_index, params):
    raise NotImplementedError("write your pallas kernel here")


# speedup vs baseline: 14.2759x; 14.2759x over previous
import functools

import jax
import jax.numpy as jnp
from jax import lax
from jax.experimental import pallas as pl
from jax.experimental.pallas import tpu as pltpu
from jax.experimental.pallas import tpu_sc as plsc

N_NODES = 10000
HID = 128
NP = 10240
NC = 2
NS = 16
NW = NC * NS
SW = NP // NS
CH = 80
BLK = 1024


def _pre_body(x_ref, wk_ref, bk_ref, wv_ref, bv_ref, av_ref, xv_ref, al_ref):
    xb = x_ref[...]
    xv_ref[...] = (
        jnp.dot(xb, wv_ref[...], preferred_element_type=jnp.float32)
        + bv_ref[...]
    )
    wa = jnp.dot(wk_ref[...], av_ref[...], preferred_element_type=jnp.float32)
    ba = jnp.dot(bk_ref[...], av_ref[...], preferred_element_type=jnp.float32)
    al_ref[...] = (
        jnp.dot(xb, wa, preferred_element_type=jnp.float32) + ba
    )


def _tc_pre(hp, p):
    ind = hp.shape[1]
    av = p['att_r'].reshape(HID, 1)
    bk = p['bK'].reshape(1, HID)
    bv = p['bV'].reshape(1, HID)
    xv, al = pl.pallas_call(
        _pre_body,
        grid=(NP // BLK,),
        in_specs=[
            pl.BlockSpec((BLK, ind), lambda i: (i, 0)),
            pl.BlockSpec((ind, HID), lambda i: (0, 0)),
            pl.BlockSpec((1, HID), lambda i: (0, 0)),
            pl.BlockSpec((ind, HID), lambda i: (0, 0)),
            pl.BlockSpec((1, HID), lambda i: (0, 0)),
            pl.BlockSpec((HID, 1), lambda i: (0, 0)),
        ],
        out_specs=[
            pl.BlockSpec((BLK, HID), lambda i: (i, 0)),
            pl.BlockSpec((BLK, 1), lambda i: (i, 0)),
        ],
        out_shape=[
            jax.ShapeDtypeStruct((NP, HID), jnp.float32),
            jax.ShapeDtypeStruct((NP, 1), jnp.float32),
        ],
    )(hp, p['WK'], bk, p['WV'], bv, av)
    return xv, al.reshape(NP)


def _ln_rows(o, g, b):
    m = jnp.mean(o, axis=-1, keepdims=True)
    v = jnp.mean((o - m) ** 2, axis=-1, keepdims=True)
    return (o - m) / jnp.sqrt(v + 1e-5) * g + b


def _post_body(p0_ref, p1_ref, ar_ref, g0_ref, b0_ref, w1_ref, c1_ref,
               w2_ref, c2_ref, g1_ref, b1_ref, out_ref):
    o = p0_ref[...] + p1_ref[...] + ar_ref[...]
    o = _ln_rows(o, g0_ref[...], b0_ref[...])
    ff = jnp.dot(
        jnp.maximum(
            jnp.dot(o, w1_ref[...], preferred_element_type=jnp.float32)
            + c1_ref[...], 0.0),
        w2_ref[...], preferred_element_type=jnp.float32) + c2_ref[...]
    o2 = o + jnp.maximum(ff, 0.0)
    out_ref[...] = jnp.maximum(_ln_rows(o2, g1_ref[...], b1_ref[...]), 0.0)


def _tc_post(p0, p1, p):
    row = lambda a: a.reshape(1, HID)
    full = lambda i: (0, 0)
    blk = lambda i: (i, 0)
    return pl.pallas_call(
        _post_body,
        grid=(NP // BLK,),
        in_specs=[
            pl.BlockSpec((BLK, HID), blk),
            pl.BlockSpec((BLK, HID), blk),
            pl.BlockSpec((1, HID), full),
            pl.BlockSpec((1, HID), full),
            pl.BlockSpec((1, HID), full),
            pl.BlockSpec((HID, HID), full),
            pl.BlockSpec((1, HID), full),
            pl.BlockSpec((HID, HID), full),
            pl.BlockSpec((1, HID), full),
            pl.BlockSpec((1, HID), full),
            pl.BlockSpec((1, HID), full),
        ],
        out_specs=pl.BlockSpec((BLK, HID), blk),
        out_shape=jax.ShapeDtypeStruct((NP, HID), jnp.float32),
    )(p0, p1, row(p['att_r'].reshape(HID)), row(p['ln0_g']), row(p['ln0_b']),
      p['W1'], row(p['b1']), p['W2'], row(p['b2']),
      row(p['ln1_g']), row(p['ln1_b']))


def _sc_softmax(alpha, sidx, didx):
    E = sidx.shape[0]
    EW = E // NW
    mesh = plsc.VectorSubcoreMesh(core_axis_name="c", subcore_axis_name="s")

    @functools.partial(
        pl.kernel,
        out_type=[
            jax.ShapeDtypeStruct((E,), jnp.float32),
            jax.ShapeDtypeStruct((NC, NP), jnp.float32),
        ],
        mesh=mesh,
        compiler_params=pltpu.CompilerParams(needs_layout_passes=False),
        scratch_types=[
            pltpu.VMEM((NP,), jnp.float32),
            pltpu.VMEM((EW,), jnp.int32),
            pltpu.VMEM((EW,), jnp.int32),
            pltpu.VMEM((EW,), jnp.float32),
            pltpu.VMEM((NP,), jnp.float32),
            pltpu.VMEM((SW,), jnp.float32),
            pltpu.VMEM((SW,), jnp.float32),
            pltpu.VMEM_SHARED((NS, NP), jnp.float32),
        ],
    )
    def k(alpha_h, sidx_h, didx_h, aexp_h, dpart_h,
          alpha_v, sidx_v, didx_v, aexp_v, dacc_v, red_v, tmp_v, sp_all):
        cid = lax.axis_index("c")
        sid = lax.axis_index("s")
        base = (cid * NS + sid) * EW
        pltpu.sync_copy(alpha_h, alpha_v)
        pltpu.sync_copy(sidx_h.at[pl.ds(base, EW)], sidx_v)
        pltpu.sync_copy(didx_h.at[pl.ds(base, EW)], didx_v)

        def zbody(i, _):
            dacc_v[pl.ds(i * 16, 16)] = jnp.zeros((16,), jnp.float32)
            return 0
        lax.fori_loop(0, NP // 16, zbody, 0, unroll=8)

        def ebody(g, _):
            s16 = sidx_v[pl.ds(g * 16, 16)]
            a16 = plsc.load_gather(alpha_v, [s16])
            a16 = jnp.where(a16 >= 0.0, a16, a16 * 0.2)
            x16 = jnp.exp(a16)
            aexp_v[pl.ds(g * 16, 16)] = x16
            d16 = didx_v[pl.ds(g * 16, 16)]
            plsc.addupdate_scatter(dacc_v, [d16], x16)
            return 0
        lax.fori_loop(0, EW // 16, ebody, 0, unroll=4)

        pltpu.sync_copy(aexp_v, aexp_h.at[pl.ds(base, EW)])
        pltpu.sync_copy(dacc_v, sp_all.at[sid])
        plsc.subcore_barrier()

        pltpu.sync_copy(sp_all.at[0, pl.ds(sid * SW, SW)], red_v)

        def rbody(t, _):
            pltpu.sync_copy(sp_all.at[t, pl.ds(sid * SW, SW)], tmp_v)

            def abody(i, _):
                red_v[pl.ds(i * 16, 16)] = (
                    red_v[pl.ds(i * 16, 16)] + tmp_v[pl.ds(i * 16, 16)])
                return 0
            lax.fori_loop(0, SW // 16, abody, 0, unroll=8)
            return 0
        lax.fori_loop(1, NS, rbody, 0)
        pltpu.sync_copy(red_v, dpart_h.at[cid, pl.ds(sid * SW, SW)])

    return k(alpha, sidx, didx)


def _sc_scatter(xv, aexp, dpart, sidx, didx):
    E = sidx.shape[0]
    EW = E // NW
    NCHK = EW // CH
    mesh = plsc.VectorSubcoreMesh(core_axis_name="c", subcore_axis_name="s")

    @functools.partial(
        pl.kernel,
        out_type=jax.ShapeDtypeStruct((NC * NP, HID), jnp.float32),
        mesh=mesh,
        compiler_params=pltpu.CompilerParams(needs_layout_passes=False),
        scratch_types=[
            pltpu.VMEM((NP,), jnp.float32),
            pltpu.VMEM((NP,), jnp.float32),
            pltpu.VMEM((EW,), jnp.float32),
            pltpu.VMEM((CH,), jnp.int32),
            pltpu.VMEM((CH,), jnp.int32),
            pltpu.VMEM((CH,), jnp.float32),
            pltpu.VMEM((CH, HID), jnp.float32),
            pltpu.VMEM_SHARED((NP, HID), jnp.float32),
            pltpu.SemaphoreType.DMA,
        ],
    )
    def k(xv_h, aexp_h, dpart_h, sidx_h, didx_h, outp_h,
          denom_v, tmp_v, aexp_v, cidx_v, dcidx_v, attn_v, rows_v,
          sp_out, gsem):
        cid = lax.axis_index("c")
        sid = lax.axis_index("s")
        base = (cid * NS + sid) * EW

        pltpu.sync_copy(dpart_h.at[0], denom_v)
        pltpu.sync_copy(dpart_h.at[1], tmp_v)

        def dbody(i, _):
            denom_v[pl.ds(i * 16, 16)] = (
                denom_v[pl.ds(i * 16, 16)] + tmp_v[pl.ds(i * 16, 16)])
            return 0
        lax.fori_loop(0, NP // 16, dbody, 0, unroll=8)

        pltpu.sync_copy(aexp_h.at[pl.ds(base, EW)], aexp_v)

        def zrow(r, _):
            for f in range(HID // 16):
                rows_v[r, pl.ds(f * 16, 16)] = jnp.zeros((16,), jnp.float32)
            return 0
        lax.fori_loop(0, CH, zrow, 0, unroll=4)
        for kk in range(SW // CH):
            pltpu.sync_copy(rows_v, sp_out.at[pl.ds(sid * SW + kk * CH, CH)])
        plsc.subcore_barrier()

        def chunk(ch, _):
            cb = base + ch * CH
            pltpu.sync_copy(sidx_h.at[pl.ds(cb, CH)], cidx_v)
            pltpu.sync_copy(didx_h.at[pl.ds(cb, CH)], dcidx_v)
            cp = pltpu.async_copy(xv_h.at[cidx_v], rows_v, gsem)

            def attnb(g, _):
                a16 = aexp_v[pl.ds(ch * CH + g * 16, 16)]
                d16 = dcidx_v[pl.ds(g * 16, 16)]
                dn16 = plsc.load_gather(denom_v, [d16])
                attn_v[pl.ds(g * 16, 16)] = a16 / (dn16 + 1e-16)
                return 0
            lax.fori_loop(0, CH // 16, attnb, 0, unroll=CH // 16)
            cp.wait()

            def sbody(j, _):
                av = plsc.load_gather(
                    attn_v, [jnp.zeros((16,), jnp.int32) + j])
                for f in range(HID // 16):
                    rows_v[j, pl.ds(f * 16, 16)] = (
                        rows_v[j, pl.ds(f * 16, 16)] * av)
                return 0
            lax.fori_loop(0, CH, sbody, 0, unroll=4)

            pltpu.sync_copy(rows_v, sp_out.at[dcidx_v], add=True)
            return 0
        lax.fori_loop(0, NCHK, chunk, 0)

        plsc.subcore_barrier()
        pltpu.sync_copy(
            sp_out.at[pl.ds(sid * SW, SW)],
            outp_h.at[pl.ds(cid * NP + sid * SW, SW)])

    return k(xv, aexp, dpart, sidx, didx)


def kernel(x, edge_index, params):
    src = edge_index[0]
    he = edge_index[1] - jnp.min(edge_index[1])
    hp = jnp.zeros((NP, x.shape[1]), x.dtype).at[:N_NODES].set(x)
    for li, p in enumerate(params):
        s_ids, d_ids = (src, he) if li % 2 == 0 else (he, src)
        xv, alpha = _tc_pre(hp, p)
        aexp, dpart = _sc_softmax(alpha, s_ids, d_ids)
        outp = _sc_scatter(xv, aexp, dpart, s_ids, d_ids)
        hp = _tc_post(outp[:NP], outp[NP:], p)
    return hp[:N_NODES]


# speedup vs baseline: 17.5516x; 1.2295x over previous
import functools

import jax
import jax.numpy as jnp
from jax import lax
from jax.experimental import pallas as pl
from jax.experimental.pallas import tpu as pltpu
from jax.experimental.pallas import tpu_sc as plsc

N_NODES = 10000
HID = 128
NP = 10240
NC = 2
NS = 16
NW = NC * NS
SW = NP // NS
CH = 80
NBUF = 4
BLK = 1024


def _pre_body(x_ref, wk_ref, bk_ref, wv_ref, bv_ref, av_ref, xv_ref, al_ref):
    xb = x_ref[...]
    xv_ref[...] = (
        jnp.dot(xb, wv_ref[...], preferred_element_type=jnp.float32)
        + bv_ref[...]
    )
    wa = jnp.dot(wk_ref[...], av_ref[...], preferred_element_type=jnp.float32)
    ba = jnp.dot(bk_ref[...], av_ref[...], preferred_element_type=jnp.float32)
    al_ref[...] = (
        jnp.dot(xb, wa, preferred_element_type=jnp.float32) + ba
    )


def _tc_pre(hp, p):
    ind = hp.shape[1]
    av = p['att_r'].reshape(HID, 1)
    bk = p['bK'].reshape(1, HID)
    bv = p['bV'].reshape(1, HID)
    xv, al = pl.pallas_call(
        _pre_body,
        grid=(NP // BLK,),
        in_specs=[
            pl.BlockSpec((BLK, ind), lambda i: (i, 0)),
            pl.BlockSpec((ind, HID), lambda i: (0, 0)),
            pl.BlockSpec((1, HID), lambda i: (0, 0)),
            pl.BlockSpec((ind, HID), lambda i: (0, 0)),
            pl.BlockSpec((1, HID), lambda i: (0, 0)),
            pl.BlockSpec((HID, 1), lambda i: (0, 0)),
        ],
        out_specs=[
            pl.BlockSpec((BLK, HID), lambda i: (i, 0)),
            pl.BlockSpec((BLK, 1), lambda i: (i, 0)),
        ],
        out_shape=[
            jax.ShapeDtypeStruct((NP, HID), jnp.float32),
            jax.ShapeDtypeStruct((NP, 1), jnp.float32),
        ],
    )(hp, p['WK'], bk, p['WV'], bv, av)
    return xv, al.reshape(NP)


def _ln_rows(o, g, b):
    m = jnp.mean(o, axis=-1, keepdims=True)
    v = jnp.mean((o - m) ** 2, axis=-1, keepdims=True)
    return (o - m) / jnp.sqrt(v + 1e-5) * g + b


def _post_body(p0_ref, p1_ref, ar_ref, g0_ref, b0_ref, w1_ref, c1_ref,
               w2_ref, c2_ref, g1_ref, b1_ref, out_ref):
    o = p0_ref[...] + p1_ref[...] + ar_ref[...]
    o = _ln_rows(o, g0_ref[...], b0_ref[...])
    ff = jnp.dot(
        jnp.maximum(
            jnp.dot(o, w1_ref[...], preferred_element_type=jnp.float32)
            + c1_ref[...], 0.0),
        w2_ref[...], preferred_element_type=jnp.float32) + c2_ref[...]
    o2 = o + jnp.maximum(ff, 0.0)
    out_ref[...] = jnp.maximum(_ln_rows(o2, g1_ref[...], b1_ref[...]), 0.0)


def _tc_post(p0, p1, p):
    row = lambda a: a.reshape(1, HID)
    full = lambda i: (0, 0)
    blk = lambda i: (i, 0)
    return pl.pallas_call(
        _post_body,
        grid=(NP // BLK,),
        in_specs=[
            pl.BlockSpec((BLK, HID), blk),
            pl.BlockSpec((BLK, HID), blk),
            pl.BlockSpec((1, HID), full),
            pl.BlockSpec((1, HID), full),
            pl.BlockSpec((1, HID), full),
            pl.BlockSpec((HID, HID), full),
            pl.BlockSpec((1, HID), full),
            pl.BlockSpec((HID, HID), full),
            pl.BlockSpec((1, HID), full),
            pl.BlockSpec((1, HID), full),
            pl.BlockSpec((1, HID), full),
        ],
        out_specs=pl.BlockSpec((BLK, HID), blk),
        out_shape=jax.ShapeDtypeStruct((NP, HID), jnp.float32),
    )(p0, p1, row(p['att_r'].reshape(HID)), row(p['ln0_g']), row(p['ln0_b']),
      p['W1'], row(p['b1']), p['W2'], row(p['b2']),
      row(p['ln1_g']), row(p['ln1_b']))


def _sc_softmax(alpha, sidx, didx):
    E = sidx.shape[0]
    EW2 = E // NS
    EH = EW2 // NC
    mesh = plsc.VectorSubcoreMesh(core_axis_name="c", subcore_axis_name="s")

    @functools.partial(
        pl.kernel,
        out_type=jax.ShapeDtypeStruct((E,), jnp.float32),
        mesh=mesh,
        compiler_params=pltpu.CompilerParams(needs_layout_passes=False),
        scratch_types=[
            pltpu.VMEM((NP,), jnp.float32),
            pltpu.VMEM((EW2,), jnp.int32),
            pltpu.VMEM((EW2,), jnp.int32),
            pltpu.VMEM((EW2,), jnp.float32),
            pltpu.VMEM((NP,), jnp.float32),
            pltpu.VMEM((SW,), jnp.float32),
            pltpu.VMEM((SW,), jnp.float32),
            pltpu.VMEM_SHARED((NS, NP), jnp.float32),
            pltpu.VMEM_SHARED((NP,), jnp.float32),
        ],
    )
    def k(alpha_h, sidx_h, didx_h, attn_h,
          alpha_v, sidx_v, didx_v, aexp_v, dacc_v, red_v, tmp_v,
          sp_all, sp_den):
        cid = lax.axis_index("c")
        sid = lax.axis_index("s")
        base = sid * EW2
        pltpu.sync_copy(alpha_h, alpha_v)
        pltpu.sync_copy(sidx_h.at[pl.ds(base, EW2)], sidx_v)
        pltpu.sync_copy(didx_h.at[pl.ds(base, EW2)], didx_v)

        def zbody(i, _):
            dacc_v[pl.ds(i * 16, 16)] = jnp.zeros((16,), jnp.float32)
            return 0
        lax.fori_loop(0, NP // 16, zbody, 0, unroll=8)

        def ebody(g, _):
            s16 = sidx_v[pl.ds(g * 16, 16)]
            a16 = plsc.load_gather(alpha_v, [s16])
            a16 = jnp.where(a16 >= 0.0, a16, a16 * 0.2)
            x16 = jnp.exp(a16)
            aexp_v[pl.ds(g * 16, 16)] = x16
            d16 = didx_v[pl.ds(g * 16, 16)]
            plsc.addupdate_scatter(dacc_v, [d16], x16)
            return 0
        lax.fori_loop(0, EW2 // 16, ebody, 0, unroll=4)

        pltpu.sync_copy(dacc_v, sp_all.at[sid])
        plsc.subcore_barrier()

        pltpu.sync_copy(sp_all.at[0, pl.ds(sid * SW, SW)], red_v)

        def rbody(t, _):
            pltpu.sync_copy(sp_all.at[t, pl.ds(sid * SW, SW)], tmp_v)

            def abody(i, _):
                red_v[pl.ds(i * 16, 16)] = (
                    red_v[pl.ds(i * 16, 16)] + tmp_v[pl.ds(i * 16, 16)])
                return 0
            lax.fori_loop(0, SW // 16, abody, 0, unroll=8)
            return 0
        lax.fori_loop(1, NS, rbody, 0)
        pltpu.sync_copy(red_v, sp_den.at[pl.ds(sid * SW, SW)])
        plsc.subcore_barrier()
        pltpu.sync_copy(sp_den, dacc_v)

        off = cid * EH

        def fbody(g, _):
            x16 = aexp_v[pl.ds(off + g * 16, 16)]
            d16 = didx_v[pl.ds(off + g * 16, 16)]
            dn16 = plsc.load_gather(dacc_v, [d16])
            aexp_v[pl.ds(off + g * 16, 16)] = x16 / (dn16 + 1e-16)
            return 0
        lax.fori_loop(0, EH // 16, fbody, 0, unroll=4)
        pltpu.sync_copy(
            aexp_v.at[pl.ds(off, EH)], attn_h.at[pl.ds(base + off, EH)])

    return k(alpha, sidx, didx)


def _sc_scatter(xv, attn, sidx, didx):
    EW = sidx.shape[0] // NW
    NCHK = EW // CH
    mesh = plsc.VectorSubcoreMesh(core_axis_name="c", subcore_axis_name="s")

    @functools.partial(
        pl.kernel,
        out_type=jax.ShapeDtypeStruct((NC * NP, HID), jnp.float32),
        mesh=mesh,
        compiler_params=pltpu.CompilerParams(needs_layout_passes=False),
        scratch_types=(
            [pltpu.VMEM((CH, HID), jnp.float32) for _ in range(NBUF)]
            + [pltpu.VMEM((CH,), jnp.int32) for _ in range(2 * NBUF)]
            + [pltpu.VMEM((CH,), jnp.float32) for _ in range(NBUF)]
            + [pltpu.VMEM_SHARED((NP, HID), jnp.float32)]
            + [pltpu.SemaphoreType.DMA for _ in range(2 * NBUF)]),
    )
    def k(xv_h, attn_h, sidx_h, didx_h, outp_h, *rest):
        rows = rest[:NBUF]
        cidx = rest[NBUF:2 * NBUF]
        didx = rest[2 * NBUF:3 * NBUF]
        attc = rest[3 * NBUF:4 * NBUF]
        sp_out = rest[4 * NBUF]
        gsem = rest[4 * NBUF + 1:4 * NBUF + 1 + NBUF]
        ssem = rest[4 * NBUF + 1 + NBUF:]
        cid = lax.axis_index("c")
        sid = lax.axis_index("s")
        base = (cid * NS + sid) * EW

        def zrow(r, _):
            for f in range(HID // 16):
                rows[0][r, pl.ds(f * 16, 16)] = jnp.zeros((16,), jnp.float32)
            return 0
        lax.fori_loop(0, CH, zrow, 0, unroll=4)
        for kk in range(SW // CH):
            pltpu.sync_copy(rows[0], sp_out.at[pl.ds(sid * SW + kk * CH, CH)])
        plsc.subcore_barrier()

        def scale(b):
            def sbody(j, _):
                av = plsc.load_gather(
                    attc[b], [jnp.zeros((16,), jnp.int32) + j])
                for f in range(HID // 16):
                    rows[b][j, pl.ds(f * 16, 16)] = (
                        rows[b][j, pl.ds(f * 16, 16)] * av)
                return 0
            lax.fori_loop(0, CH, sbody, 0, unroll=4)

        def body(p, _):
            ch0 = p * NBUF
            gds = []
            for b in range(NBUF):
                cb = base + (ch0 + b) * CH
                pltpu.sync_copy(sidx_h.at[pl.ds(cb, CH)], cidx[b])
                pltpu.sync_copy(didx_h.at[pl.ds(cb, CH)], didx[b])
                pltpu.sync_copy(attn_h.at[pl.ds(cb, CH)], attc[b])
                gds.append(pltpu.async_copy(
                    xv_h.at[cidx[b]], rows[b], gsem[b]))
            sds = []
            for b in range(NBUF):
                gds[b].wait()
                scale(b)
                sds.append(pltpu.async_copy(
                    rows[b], sp_out.at[didx[b]], ssem[b], add=True))
            for d in sds:
                d.wait()
            return 0
        lax.fori_loop(0, NCHK // NBUF, body, 0)
        for ch in range(NCHK - NCHK % NBUF, NCHK):
            cb = base + ch * CH
            pltpu.sync_copy(sidx_h.at[pl.ds(cb, CH)], cidx[0])
            pltpu.sync_copy(didx_h.at[pl.ds(cb, CH)], didx[0])
            pltpu.sync_copy(attn_h.at[pl.ds(cb, CH)], attc[0])
            pltpu.async_copy(xv_h.at[cidx[0]], rows[0], gsem[0]).wait()
            scale(0)
            pltpu.sync_copy(rows[0], sp_out.at[didx[0]], add=True)

        plsc.subcore_barrier()
        pltpu.sync_copy(
            sp_out.at[pl.ds(sid * SW, SW)],
            outp_h.at[pl.ds(cid * NP + sid * SW, SW)])

    return k(xv, attn, sidx, didx)


def kernel(x, edge_index, params):
    src = edge_index[0]
    he = edge_index[1] - jnp.min(edge_index[1])
    hp = jnp.zeros((NP, x.shape[1]), x.dtype).at[:N_NODES].set(x)
    for li, p in enumerate(params):
        s_ids, d_ids = (src, he) if li % 2 == 0 else (he, src)
        xv, alpha = _tc_pre(hp, p)
        attn = _sc_softmax(alpha, s_ids, d_ids)
        outp = _sc_scatter(xv, attn, s_ids, d_ids)
        hp = _tc_post(outp[:NP], outp[NP:], p)
    return hp[:N_NODES]
